# baseline (speedup 1.0000x reference)
import functools

import jax
import jax.numpy as jnp
from jax import lax
from jax.experimental import pallas as pl
from jax.experimental.pallas import tpu as pltpu
from jax.experimental.pallas import tpu_sc as plsc

_NC = 2
_NS = 16
_NW = _NC * _NS

_TOTAL = 4 * 4096 * 2048
_PER_W = _TOTAL // _NW
_CH = 32768
_NCH = _PER_W // _CH
_NG = _NCH // 2

_mesh = plsc.VectorSubcoreMesh(core_axis_name="c", subcore_axis_name="s")


@functools.partial(
    pl.kernel,
    mesh=_mesh,
    out_type=jax.ShapeDtypeStruct((_TOTAL,), jnp.float32),
    scratch_types=[
        pltpu.VMEM((_CH,), jnp.float32),
        pltpu.VMEM((_CH,), jnp.float32),
        pltpu.SemaphoreType.DMA,
        pltpu.SemaphoreType.DMA,
        pltpu.SemaphoreType.DMA,
        pltpu.SemaphoreType.DMA,
    ],
)
def _sc_copy(x_hbm, o_hbm, buf0, buf1, isem0, isem1, osem0, osem1):
    wid = lax.axis_index("s") * _NC + lax.axis_index("c")
    base = wid * _PER_W
    bufs = (buf0, buf1)
    isems = (isem0, isem1)
    osems = (osem0, osem1)

    def in_cp(idx, b):
        return pltpu.make_async_copy(
            x_hbm.at[pl.ds(base + idx * _CH, _CH)], bufs[b], isems[b])

    def out_cp(idx, b):
        return pltpu.make_async_copy(
            bufs[b], o_hbm.at[pl.ds(base + idx * _CH, _CH)], osems[b])

    in_cp(0, 0).start()

    def group(g, carry):
        i0 = g * 2
        in_cp(i0, 0).wait()
        out_cp(i0, 0).start()

        @pl.when(g > 0)
        def _():
            out_cp(i0 - 1, 1).wait()

        in_cp(i0 + 1, 1).start()
        in_cp(i0 + 1, 1).wait()
        out_cp(i0 + 1, 1).start()
        out_cp(i0, 0).wait()

        @pl.when(g < _NG - 1)
        def _():
            in_cp(i0 + 2, 0).start()

        return carry

    lax.fori_loop(0, _NG, group, 0)
    out_cp(_NCH - 1, 1).wait()


def kernel(x):
    b, s, d = x.shape
    out = _sc_copy(x.reshape(-1))
    return out.reshape(b, s, d)
